# baseline (speedup 1.0000x reference)
import functools
import math

import jax
import jax.numpy as jnp
from jax import lax
from jax.experimental import pallas as pl
from jax.experimental.pallas import tpu as pltpu
from jax.experimental.pallas import tpu_sc as plsc

CUTOFF = 5.0
EPS = 1e-8
H = 8
_NC = 2
_NS = 16


def _sc_gather_quad(tabs, idxs, C=40):
    n_workers = _NC * _NS
    E = idxs[0].shape[0]
    PW = E // (n_workers // 2)
    NCH = PW // C
    widths = [t.shape[1] for t in tabs]
    assert widths[0] == widths[1] and widths[2] == widths[3]
    mesh = plsc.VectorSubcoreMesh(core_axis_name="c", subcore_axis_name="s")

    @functools.partial(
        pl.kernel, mesh=mesh,
        out_type=[jax.ShapeDtypeStruct((E, w), jnp.float32) for w in widths],
        scratch_types=[
            pltpu.VMEM((PW,), jnp.int32),
            pltpu.VMEM((C, widths[0]), jnp.float32),
            pltpu.VMEM((C, widths[0]), jnp.float32),
            pltpu.VMEM((C, widths[2]), jnp.float32),
            pltpu.VMEM((C, widths[2]), jnp.float32),
            pltpu.SemaphoreType.DMA,
            pltpu.SemaphoreType.DMA,
        ],
    )
    def gk(tab0_h, tab1_h, tab2_h, tab3_h, idx0_h, idx1_h, idx2_h, idx3_h,
           out0_h, out1_h, out2_h, out3_h,
           idx_v, bufa0, bufa1, bufb0, bufb1, sem0, sem1):
        wid = lax.axis_index("s") * _NC + lax.axis_index("c")
        tid = wid % 2
        base = (wid // 2) * PW

        def pipe(tab_h, idxh, out_h, buf0, buf1):
            pltpu.sync_copy(idxh.at[pl.ds(base, PW)], idx_v)

            def start(j, buf, sem):
                off = pl.multiple_of(j * C, 8)
                pltpu.make_async_copy(
                    tab_h.at[idx_v.at[pl.ds(off, C)]], buf, sem).start()

            def wait(buf, sem):
                pltpu.make_async_copy(
                    tab_h.at[idx_v.at[pl.ds(0, C)]], buf, sem).wait()

            def writeout(j, buf):
                pltpu.sync_copy(buf, out_h.at[pl.ds(base + j * C, C)])

            start(0, buf0, sem0)

            def body(g, carry):
                j0 = g * 2
                start(j0 + 1, buf1, sem1)
                wait(buf0, sem0)
                writeout(j0, buf0)

                @pl.when(j0 + 2 < NCH)
                def _():
                    start(j0 + 2, buf0, sem0)

                wait(buf1, sem1)
                writeout(j0 + 1, buf1)
                return carry

            lax.fori_loop(0, NCH // 2, body, 0)

        @pl.when(tid == 0)
        def _():
            pipe(tab0_h, idx0_h, out0_h, bufa0, bufa1)

        @pl.when(tid == 1)
        def _():
            pipe(tab1_h, idx1_h, out1_h, bufa0, bufa1)

        @pl.when(tid == 0)
        def _():
            pipe(tab2_h, idx2_h, out2_h, bufb0, bufb1)

        @pl.when(tid == 1)
        def _():
            pipe(tab3_h, idx3_h, out3_h, bufb0, bufb1)

    return gk(*tabs, *idxs)


def _sc_scatter_add(payload, dst, N, init=None, CW=128):
    E, P = payload.shape
    n_chunks = P // CW
    rpc = n_chunks // _NC
    PW = E // _NS
    W = 80 if PW % 80 == 0 else 40
    NIT = PW // W
    RPS = -(-N // (_NS * 8)) * 8
    Np = RPS * _NS
    zeros = jnp.zeros((RPS, CW), jnp.float32)
    dst3 = dst.reshape(_NS, NIT, W)
    mesh = plsc.VectorSubcoreMesh(core_axis_name="c", subcore_axis_name="s")
    with_init = init is not None

    @functools.partial(
        pl.kernel, mesh=mesh,
        out_type=jax.ShapeDtypeStruct((Np, P), jnp.float32),
        scratch_types=[
            pltpu.VMEM((W, CW), jnp.float32),
            pltpu.VMEM((W, CW), jnp.float32),
            pltpu.VMEM((NIT, W), jnp.int32),
            pltpu.VMEM_SHARED((Np, CW), jnp.float32),
            pltpu.SemaphoreType.DMA,
            pltpu.SemaphoreType.DMA,
        ],
    )
    def sk(pay_h, dst3_h, z_h, out_h, pay0, pay1, idx_all, accum, sem0, sem1):
        cid = lax.axis_index("c")
        sid = lax.axis_index("s")
        rows = pl.ds(sid * RPS, RPS)
        pltpu.sync_copy(dst3_h.at[sid], idx_all)

        for r in range(rpc):
            coloff = pl.multiple_of((cid * rpc + r) * CW, CW)
            if with_init:
                pltpu.sync_copy(z_h.at[rows, pl.ds(coloff, CW)],
                                accum.at[rows])
            else:
                pltpu.sync_copy(z_h, accum.at[rows])
            plsc.subcore_barrier()

            def start_load(i, buf, sem):
                e0 = pl.multiple_of(sid * PW + i * W, 8)
                pltpu.make_async_copy(
                    pay_h.at[pl.ds(e0, W), pl.ds(coloff, CW)], buf,
                    sem).start()

            def step(i, buf, sem, obuf, osem):
                @pl.when(i + 1 < NIT)
                def _():
                    start_load(i + 1, obuf, osem)

                pltpu.make_async_copy(
                    pay_h.at[pl.ds(0, W), pl.ds(0, CW)], buf, sem).wait()
                pltpu.sync_copy(buf, accum.at[idx_all.at[i]], add=True)

            start_load(0, pay0, sem0)

            def body(i, carry):
                @pl.when(i % 2 == 0)
                def _():
                    step(i, pay0, sem0, pay1, sem1)

                @pl.when(i % 2 == 1)
                def _():
                    step(i, pay1, sem1, pay0, sem0)

                return carry

            lax.fori_loop(0, NIT, body, 0)
            plsc.subcore_barrier()
            pltpu.sync_copy(accum.at[rows],
                            out_h.at[rows, pl.ds(coloff, CW)])
            plsc.subcore_barrier()

    return sk(payload, dst3, init if with_init else zeros)


def _pack_bf16(x):
    W = x.shape[1] // 2
    xr = x.astype(jnp.bfloat16).astype(jnp.float32)
    bits = lax.bitcast_convert_type(xr, jnp.uint32)
    word = bits[:, :W] | (bits[:, W:] >> 16)
    return lax.bitcast_convert_type(word, jnp.float32)


def _unpack2_bf16(x):
    w = lax.bitcast_convert_type(x, jnp.uint32)
    a = lax.bitcast_convert_type(w & jnp.uint32(0xFFFF0000), jnp.float32)
    b = lax.bitcast_convert_type(w << 16, jnp.float32)
    return a, b


def _unpack_bf16(x):
    a, b = _unpack2_bf16(x)
    return jnp.concatenate([a, b], axis=1)


def _node_body(s_ref, t_ref, wq, bq, wk, bk, g1, b1, g2, b2, v1, c1, v2, c2,
               wvq, wvk, dst_tab_ref, src_a_ref, ts_ref, tk_ref, u_ref):
    F = s_ref.shape[1]
    Fh = F // 2
    s = s_ref[...]
    t768 = t_ref[...]
    q = jnp.dot(s, wq[...], preferred_element_type=jnp.float32) + bq[...]
    k = jnp.dot(s, wk[...], preferred_element_type=jnp.float32) + bk[...]
    h1 = jnp.dot(s, g1[...], preferred_element_type=jnp.float32) + b1[...]
    h1 = h1 * jax.nn.sigmoid(h1)
    xs = jnp.dot(h1, g2[...], preferred_element_type=jnp.float32) + b2[...]
    h2 = jnp.dot(s, v1[...], preferred_element_type=jnp.float32) + c1[...]
    h2 = h2 * jax.nn.sigmoid(h2)
    u = jnp.dot(h2, v2[...], preferred_element_type=jnp.float32) + c2[...]
    ts_ref[...] = _pack_bf16(t768)
    u_ref[...] = u
    tqs = [jnp.dot(t768[:, l * F:(l + 1) * F], wvq[...],
                   preferred_element_type=jnp.float32) for l in range(3)]
    tks = [jnp.dot(t768[:, l * F:(l + 1) * F], wvk[...],
                   preferred_element_type=jnp.float32) for l in range(3)]
    dst_tab_ref[...] = _pack_bf16(jnp.concatenate([q] + tqs, axis=1))
    src_a_ref[...] = _pack_bf16(jnp.concatenate([k, xs], axis=1))
    tk_ref[...] = _pack_bf16(jnp.concatenate(tks, axis=1))


def _node_stage(s, t768, Wq_w, Wq_b, Wk_w, Wk_b, gs1_w, gs1_b, gs2_w, gs2_b,
                gv1_w, gv1_b, gv2_w, gv2_b, Wvq_w, Wvk0_w, block_n=1000):
    N, F = s.shape
    grid = (N // block_n,)
    row = lambda i: (i, 0)
    full = lambda shape: pl.BlockSpec(shape, lambda i: (0, 0))
    in_specs = [
        pl.BlockSpec((block_n, F), row),
        pl.BlockSpec((block_n, 3 * F), row),
        full((F, F)), pl.BlockSpec((F,), lambda i: (0,)),
        full((F, F)), pl.BlockSpec((F,), lambda i: (0,)),
        full((F, F)), pl.BlockSpec((F,), lambda i: (0,)),
        full((F, 3 * F)), pl.BlockSpec((3 * F,), lambda i: (0,)),
        full((F, F)), pl.BlockSpec((F,), lambda i: (0,)),
        full((F, 3 * F)), pl.BlockSpec((3 * F,), lambda i: (0,)),
        full((F, F)), full((F, F)),
    ]
    out_specs = [
        pl.BlockSpec((block_n, 2 * F), row),
        pl.BlockSpec((block_n, 2 * F), row),
        pl.BlockSpec((block_n, 3 * F // 2), row),
        pl.BlockSpec((block_n, 3 * F // 2), row),
        pl.BlockSpec((block_n, 3 * F), row),
    ]
    out_shape = [
        jax.ShapeDtypeStruct((N, 2 * F), jnp.float32),
        jax.ShapeDtypeStruct((N, 2 * F), jnp.float32),
        jax.ShapeDtypeStruct((N, 3 * F // 2), jnp.float32),
        jax.ShapeDtypeStruct((N, 3 * F // 2), jnp.float32),
        jax.ShapeDtypeStruct((N, 3 * F), jnp.float32),
    ]
    return pl.pallas_call(
        _node_body, grid=grid, in_specs=in_specs, out_specs=out_specs,
        out_shape=out_shape,
    )(s, t768, Wq_w, Wq_b, Wk_w, Wk_b, gs1_w, gs1_b, gs2_w, gs2_b,
      gv1_w, gv1_b, gv2_w, gv2_b, Wvq_w, Wvk0_w)


def _edge_body(r_ref, gdst_ref, gsa_ref, gts_ref, gtk_ref, dir_ref, d_ref,
               ne_ref, wre, bre, wrs, brs, wgt, bgt, payload_ref, rout_ref):
    F = r_ref.shape[1]
    dh = F // H
    B = r_ref.shape[0]
    r = r_ref[...]
    ga, gb = _unpack2_bf16(gdst_ref[...])
    qd = ga[:, :F]
    tq = (ga[:, F:], gb[:, :F], gb[:, F:])
    sa, sb = _unpack2_bf16(gsa_ref[...])
    ks = sa[:, :F]
    xs3 = (sa[:, F:], sb[:, :F], sb[:, F:])
    ts = _unpack_bf16(gts_ref[...])
    tks = _unpack_bf16(gtk_ref[...])
    d_ij = d_ref[...]
    nume = ne_ref[...]

    ra = jnp.dot(r, wre[...], preferred_element_type=jnp.float32) + bre[...]
    prod = qd * ks * ra

    rows = lax.broadcasted_iota(jnp.int32, (F, H), 0)
    cols = lax.broadcasted_iota(jnp.int32, (F, H), 1)
    mhead = (rows // dh == cols).astype(jnp.float32)
    attn = jnp.dot(prod, mhead, preferred_element_type=jnp.float32)
    attn = attn * jax.nn.sigmoid(attn)
    cut = 0.5 * (jnp.cos(d_ij * (math.pi / CUTOFF)) + 1.0)
    cut = cut * (d_ij < CUTOFF).astype(jnp.float32)
    invn = lax.rsqrt(jnp.maximum(nume, 1.0))
    attn = attn * (cut * invn)

    hrow = lax.broadcasted_iota(jnp.int32, (H, 3 * F), 0)
    hcol = lax.broadcasted_iota(jnp.int32, (H, 3 * F), 1)
    mb = (hrow == hcol // (3 * dh)).astype(jnp.float32)
    scale = jnp.dot(attn, mb, preferred_element_type=jnp.float32)

    rs = jnp.dot(r, wrs[...], preferred_element_type=jnp.float32) + brs[...]
    o_s = xs3[0] * rs[:, :F] * scale[:, :F]
    o_d = xs3[1] * rs[:, F:2 * F] * scale[:, F:2 * F]
    o_t = xs3[2] * rs[:, 2 * F:] * scale[:, 2 * F:]

    payload_ref[:, :F] = o_s
    for l in range(3):
        dl = dir_ref[:, l:l + 1]
        payload_ref[:, F + l * F:F + (l + 1) * F] = (
            o_d * dl + o_t * ts[:, l * F:(l + 1) * F])

    d0 = dir_ref[:, 0:1]
    d1 = dir_ref[:, 1:2]
    d2 = dir_ref[:, 2:3]
    dnorm = d0 * d0 + d1 * d1 + d2 * d2 + EPS
    a = tq[0] * d0 + tq[1] * d1 + tq[2] * d2
    b = tks[:, :F] * d0 + tks[:, F:2 * F] * d1 + tks[:, 2 * F:] * d2
    tdot = (tq[0] * tks[:, :F] + tq[1] * tks[:, F:2 * F]
            + tq[2] * tks[:, 2 * F:])
    w_dot = tdot - a * b / dnorm

    df = jnp.dot(r, wgt[...], preferred_element_type=jnp.float32) + bgt[...]
    rout_ref[...] = r + df * w_dot * cut


def _edge_stage(r_ij, gdst, gsa, gts, gtk, dir_ij, d2, ne2,
                Wre_w, Wre_b, Wrs_w, Wrs_b, gt1_w, gt1_b, block_e=1000):
    E, F = r_ij.shape
    grid = (E // block_e,)
    row = lambda i: (i, 0)
    full = lambda shape: pl.BlockSpec(shape, lambda i: (0, 0))
    in_specs = [
        pl.BlockSpec((block_e, F), row),
        pl.BlockSpec((block_e, 2 * F), row),
        pl.BlockSpec((block_e, 2 * F), row),
        pl.BlockSpec((block_e, 3 * F // 2), row),
        pl.BlockSpec((block_e, 3 * F // 2), row),
        pl.BlockSpec((block_e, 3), row),
        pl.BlockSpec((block_e, 1), row),
        pl.BlockSpec((block_e, 1), row),
        full((F, F)), pl.BlockSpec((F,), lambda i: (0,)),
        full((F, 3 * F)), pl.BlockSpec((3 * F,), lambda i: (0,)),
        full((F, F)), pl.BlockSpec((F,), lambda i: (0,)),
    ]
    out_specs = [
        pl.BlockSpec((block_e, 4 * F), row),
        pl.BlockSpec((block_e, F), row),
    ]
    out_shape = [
        jax.ShapeDtypeStruct((E, 4 * F), jnp.float32),
        jax.ShapeDtypeStruct((E, F), jnp.float32),
    ]
    return pl.pallas_call(
        _edge_body, grid=grid, in_specs=in_specs, out_specs=out_specs,
        out_shape=out_shape,
    )(r_ij, gdst, gsa, gts, gtk, dir_ij, d2, ne2,
      Wre_w, Wre_b, Wrs_w, Wrs_b, gt1_w, gt1_b)


def _final_body(s_ref, t_ref, acc_ref, u_ref, sout_ref, tout_ref):
    F = s_ref.shape[1]
    s = s_ref[...]
    t768 = t_ref[...]
    ds = acc_ref[:, :F]
    dt = acc_ref[:, F:]
    u = u_ref[...]
    u1 = u[:, :F]
    u2 = u[:, F:2 * F]
    u3 = u[:, 2 * F:]
    tn = jnp.sqrt(dt[:, :F] ** 2 + dt[:, F:2 * F] ** 2 + dt[:, 2 * F:] ** 2
                  + EPS)
    sout_ref[...] = s + ds + u1 * tn + u3
    for l in range(3):
        dtl = dt[:, l * F:(l + 1) * F]
        tout_ref[:, l * F:(l + 1) * F] = (
            t768[:, l * F:(l + 1) * F] + dtl + u2 * dtl)


def _final_stage(s, t768, acc, u, block_n=1000):
    N, F = s.shape
    grid = (N // block_n,)
    row = lambda i: (i, 0)
    in_specs = [
        pl.BlockSpec((block_n, F), row),
        pl.BlockSpec((block_n, 3 * F), row),
        pl.BlockSpec((block_n, 4 * F), row),
        pl.BlockSpec((block_n, 3 * F), row),
    ]
    out_specs = [
        pl.BlockSpec((block_n, F), row),
        pl.BlockSpec((block_n, 3 * F), row),
    ]
    out_shape = [
        jax.ShapeDtypeStruct((N, F), jnp.float32),
        jax.ShapeDtypeStruct((N, 3 * F), jnp.float32),
    ]
    return pl.pallas_call(
        _final_body, grid=grid, in_specs=in_specs, out_specs=out_specs,
        out_shape=out_shape,
    )(s, t768, acc, u)


def kernel(edge_index, s, t, dir_ij, r_ij, d_ij, num_edges_expanded,
           Wq_w, Wq_b, Wk_w, Wk_b, gs1_w, gs1_b, gs2_w, gs2_b,
           gv1_w, gv1_b, gv2_w, gv2_b, Wre_w, Wre_b, Wrs_w, Wrs_b,
           gt1_w, gt1_b, Wvq_w, Wvk0_w):
    N, F = s.shape
    E = r_ij.shape[0]
    t768 = t.reshape(N, 3 * F)
    src = edge_index[0]
    dst = edge_index[1]

    dst_tab, src_a, ts_p, tk_p, u = _node_stage(
        s, t768, Wq_w, Wq_b, Wk_w, Wk_b, gs1_w, gs1_b, gs2_w, gs2_b,
        gv1_w, gv1_b, gv2_w, gv2_b, Wvq_w, Wvk0_w)

    gdst, gsa, gts, gtk = _sc_gather_quad(
        (dst_tab, src_a, ts_p, tk_p), (dst, src, src, src), C=40)

    payload, r_out = _edge_stage(
        r_ij, gdst, gsa, gts, gtk, dir_ij, d_ij[:, None],
        num_edges_expanded[:, None],
        Wre_w, Wre_b, Wrs_w, Wrs_b, gt1_w, gt1_b)

    acc = _sc_scatter_add(payload, dst, N)

    s_out, t_out768 = _final_stage(s, t768, acc, u)
    return (s_out, t_out768.reshape(N, 3, F), r_out)
